# Initial kernel scaffold; baseline (speedup 1.0000x reference)
import jax
import jax.numpy as jnp
from jax.experimental import pallas as pl


def kernel(x, edge_index, edge_attr, batch, W, b, W2, b2):
    raise NotImplementedError("write your pallas kernel here")


# speedup vs baseline: 96.6795x; 96.6795x over previous
import functools

import jax
import jax.numpy as jnp
from jax import lax
from jax.experimental import pallas as pl
from jax.experimental.pallas import tpu as pltpu
from jax.experimental.pallas import tpu_sc as plsc

_N = 100000
_E = 6400000
_G = 64
_NP = 100352
_S = _NP // 16
_ROWS = _E // 128
_CH_ROWS = 8
_NCHUNK = _ROWS // _CH_ROWS
_NW = 32
_Q, _R = divmod(_NCHUNK, _NW)
_SC = 784

_mesh = plsc.VectorSubcoreMesh(core_axis_name="c", subcore_axis_name="s",
                               num_cores=2, num_subcores=16)


def _rsqrt16(d):
    i = plsc.bitcast(d, jnp.int32)
    i = jnp.int32(0x5F3759DF) - lax.shift_right_logical(i, 1)
    r = plsc.bitcast(i, jnp.float32)
    for _ in range(5):
        r = r * (1.5 - 0.5 * d * r * r)
    return r


def _worker_range(w):
    start = w * _Q + jnp.minimum(w, _R)
    cnt = _Q + (w < _R).astype(jnp.int32)
    return start, cnt


@functools.partial(
    pl.kernel,
    out_type=pltpu.HBM((2, _NP), jnp.float32),
    mesh=_mesh,
    compiler_params=pltpu.CompilerParams(needs_layout_passes=False, use_tc_tiling_on_sc=False),
    scratch_types=[
        pltpu.VMEM((_CH_ROWS, 128), jnp.int32),
        pltpu.VMEM((_CH_ROWS, 128), jnp.float32),
        pltpu.VMEM_SHARED((_NP,), jnp.float32),
        pltpu.SemaphoreType.DMA,
        pltpu.SemaphoreType.DMA,
    ],
)
def _deg_kernel(col2d, ew2d, z1, deg2, col_v, ew_v, deg_sh, sem, asem):
    c = lax.axis_index("c")
    s = lax.axis_index("s")
    base = s * _S
    pltpu.sync_copy(z1, deg_sh.at[pl.ds(base, _S)])
    plsc.subcore_barrier()

    w = c * 16 + s
    start, cnt = _worker_range(w)

    def chunk(t, carry):
        ci = start + t
        dc = pltpu.async_copy(col2d.at[pl.ds(ci * _CH_ROWS, _CH_ROWS), :],
                              col_v, sem)
        de = pltpu.async_copy(ew2d.at[pl.ds(ci * _CH_ROWS, _CH_ROWS), :],
                              ew_v, sem)
        dc.wait()
        de.wait()
        descs = [
            pltpu.async_copy(ew_v.at[j], deg_sh.at[col_v.at[j]], asem,
                             add=True)
            for j in range(_CH_ROWS)
        ]
        for d in descs:
            d.wait()
        return carry

    lax.fori_loop(0, cnt, chunk, 0)
    plsc.subcore_barrier()
    pltpu.sync_copy(deg_sh.at[pl.ds(base, _S)], deg2.at[c, pl.ds(base, _S)])


_DBLK = _NP // 16


def _prescale_body(deg_ref, x_ref, w_ref, y_ref):
    deg = deg_ref[0] + deg_ref[1] + 1.0
    dis = lax.rsqrt(deg)
    xwt = (w_ref[:, 0][:, None] * x_ref[:, 0][None, :]
           + w_ref[:, 1][:, None] * x_ref[:, 1][None, :])
    y_ref[...] = dis[None, :] * xwt


_prescale_call = pl.pallas_call(
    _prescale_body,
    grid=(16,),
    in_specs=[
        pl.BlockSpec((2, _DBLK), lambda i: (0, i)),
        pl.BlockSpec((_DBLK, 2), lambda i: (i, 0)),
        pl.BlockSpec((4, 2), lambda i: (0, 0)),
    ],
    out_specs=pl.BlockSpec((4, _DBLK), lambda i: (0, i)),
    out_shape=jax.ShapeDtypeStruct((4, _NP), jnp.float32),
)


@functools.partial(
    pl.kernel,
    out_type=pltpu.HBM((2, 4, _NP), jnp.float32),
    mesh=_mesh,
    compiler_params=pltpu.CompilerParams(needs_layout_passes=False, use_tc_tiling_on_sc=False),
    scratch_types=[
        pltpu.VMEM((_CH_ROWS, 128), jnp.int32),
        pltpu.VMEM((_CH_ROWS, 128), jnp.int32),
        pltpu.VMEM((_CH_ROWS * 128,), jnp.float32),
        pltpu.VMEM((4, 128), jnp.float32),
        pltpu.VMEM((4, 128), jnp.float32),
        pltpu.VMEM_SHARED((4, _NP), jnp.float32),
        pltpu.VMEM_SHARED((4, _NP), jnp.float32),
        pltpu.SemaphoreType.DMA,
        pltpu.SemaphoreType.DMA,
        pltpu.SemaphoreType.DMA,
    ],
)
def _msg_kernel(row2d, col2d, ewf, yf, z1, acc2,
                row_v, col_v, ew_v,
                mi_v, mo_v, y_sh, acc_sh, sem, gsem, asem):
    c = lax.axis_index("c")
    s = lax.axis_index("s")
    base = s * _S

    for ch in range(4):
        pltpu.sync_copy(z1, acc_sh.at[ch, pl.ds(base, _S)])
        pltpu.sync_copy(yf.at[ch, pl.ds(base, _S)],
                        y_sh.at[ch, pl.ds(base, _S)])
    plsc.subcore_barrier()

    w = c * 16 + s
    start, cnt = _worker_range(w)

    def chunk(t, carry):
        ci = start + t
        dr = pltpu.async_copy(row2d.at[pl.ds(ci * _CH_ROWS, _CH_ROWS), :],
                              row_v, sem)
        dc = pltpu.async_copy(col2d.at[pl.ds(ci * _CH_ROWS, _CH_ROWS), :],
                              col_v, sem)
        de = pltpu.async_copy(ewf.at[pl.ds(ci * _CH_ROWS * 128,
                                           _CH_ROWS * 128)], ew_v, sem)
        dr.wait()
        dc.wait()
        de.wait()

        def row_body(j, carry2):
            gds = [
                pltpu.async_copy(y_sh.at[ch].at[row_v.at[j]],
                                 mi_v.at[ch], gsem)
                for ch in range(4)
            ]
            for g in gds:
                g.wait()

            def mul_body(k, carry3):
                sl = pl.ds(k * 16, 16)
                w16 = ew_v[pl.ds(j * 128 + k * 16, 16)]
                for ch in range(4):
                    mo_v[ch, sl] = mi_v[ch, sl] * w16
                return carry3

            lax.fori_loop(0, 8, mul_body, 0)
            ads = [
                pltpu.async_copy(mo_v.at[ch],
                                 acc_sh.at[ch].at[col_v.at[j]], asem,
                                 add=True)
                for ch in range(4)
            ]
            for a in ads:
                a.wait()
            return carry2

        lax.fori_loop(0, _CH_ROWS, row_body, 0)
        return carry

    lax.fori_loop(0, cnt, chunk, 0)
    plsc.subcore_barrier()
    for ch in range(4):
        pltpu.sync_copy(acc_sh.at[ch, pl.ds(base, _S)],
                        acc2.at[c, ch, pl.ds(base, _S)])


def _final_body(acc_ref, deg_ref, x_ref, bt_ref, w_ref, b_ref, w2_ref,
                b2_ref, out_ref):
    i = pl.program_id(0)
    acc = acc_ref[0] + acc_ref[1]
    deg = deg_ref[0] + deg_ref[1] + 1.0
    dis = lax.rsqrt(deg)
    xwt = (w_ref[:, 0][:, None] * x_ref[:, 0][None, :]
           + w_ref[:, 1][:, None] * x_ref[:, 1][None, :])
    agg = dis[None, :] * (acc + dis[None, :] * xwt)
    h = jnp.maximum(agg + b_ref[0][:, None], 0.0)
    o = jnp.sum(h * w2_ref[0][:, None], axis=0) + b2_ref[0, 0]
    gid = lax.broadcasted_iota(jnp.int32, (_DBLK,), 0) + i * _DBLK
    o = jnp.where(gid < _N, o, 0.0)
    bt = bt_ref[0, 0]
    oh = (lax.broadcasted_iota(jnp.int32, (_G, _DBLK), 0)
          == bt[None, :]).astype(jnp.float32)
    contrib = jnp.sum(oh * o[None, :], axis=1)

    @pl.when(i == 0)
    def _():
        out_ref[...] = jnp.zeros_like(out_ref)

    out_ref[...] += contrib[:, None]


_final_call = pl.pallas_call(
    _final_body,
    grid=(16,),
    in_specs=[
        pl.BlockSpec((2, 4, _DBLK), lambda i: (0, 0, i)),
        pl.BlockSpec((2, _DBLK), lambda i: (0, i)),
        pl.BlockSpec((_DBLK, 2), lambda i: (i, 0)),
        pl.BlockSpec((1, 1, _DBLK), lambda i: (i, 0, 0)),
        pl.BlockSpec((4, 2), lambda i: (0, 0)),
        pl.BlockSpec((1, 4), lambda i: (0, 0)),
        pl.BlockSpec((1, 4), lambda i: (0, 0)),
        pl.BlockSpec((1, 1), lambda i: (0, 0)),
    ],
    out_specs=pl.BlockSpec((_G, 1), lambda i: (0, 0)),
    out_shape=jax.ShapeDtypeStruct((_G, 1), jnp.float32),
)


def kernel(x, edge_index, edge_attr, batch, W, b, W2, b2):
    row2d = edge_index[0].reshape(_ROWS, 128)
    col2d = edge_index[1].reshape(_ROWS, 128)
    ew2d = edge_attr.reshape(_ROWS, 128)
    x_pad = jnp.zeros((_NP, 2), jnp.float32).at[:_N].set(x)
    z1 = jnp.zeros((_S,), jnp.float32)

    deg2 = _deg_kernel(col2d, ew2d, z1)
    yf = _prescale_call(deg2, x_pad, W)
    acc2 = _msg_kernel(row2d, col2d, edge_attr, yf, z1)

    bt3 = (jnp.zeros((_NP,), jnp.int32).at[:_N].set(batch)
           .reshape(16, 1, _DBLK))
    pooled = _final_call(acc2, deg2, x_pad, bt3, W, b.reshape(1, 4),
                         W2, b2.reshape(1, 1))
    return pooled
